# Initial kernel scaffold; baseline (speedup 1.0000x reference)
import jax
import jax.numpy as jnp
from jax.experimental import pallas as pl


def kernel(x, edge_index):
    raise NotImplementedError("write your pallas kernel here")


# speedup vs baseline: 2.3425x; 2.3425x over previous
import functools

import jax
import jax.numpy as jnp
from jax import lax
from jax.experimental import pallas as pl
from jax.experimental.pallas import tpu as pltpu
from jax.experimental.pallas import tpu_sc as plsc

_NC = 2
_NS = 16
_NW = _NC * _NS

_N_NODES = 10000
_D = 128
_N_EDGES = 320000
_B = 2 * _N_EDGES
_PER_W = _B // _NW
_C = 80
_NCHUNK = _PER_W // _C
_NBUF = 5
_OUTER = _NCHUNK // _NBUF


def _gather_body(x_hbm, idx_hbm, out_hbm, idx_v, buf, *sems):
    gsems = sems[:_NBUF]
    wsems = sems[_NBUF:]
    wid = lax.axis_index("s") * _NC + lax.axis_index("c")
    base = wid * _PER_W

    pltpu.sync_copy(idx_hbm.at[wid], idx_v)

    def gather_start(j, b):
        pltpu.async_copy(x_hbm.at[idx_v.at[j]], buf.at[b], gsems[b])

    def gather_wait(b):
        pltpu.make_async_copy(x_hbm, buf.at[b], gsems[b]).wait()

    def write_start(j, b):
        pltpu.async_copy(buf.at[b], out_hbm.at[pl.ds(base + j * _C, _C)],
                         wsems[b])

    def write_wait(j, b):
        pltpu.make_async_copy(buf.at[b],
                              out_hbm.at[pl.ds(base + j * _C, _C)],
                              wsems[b]).wait()

    for j in range(_NBUF - 1):
        gather_start(j, j)

    def outer(jh, _):
        for b in range(_NBUF):
            j = jh * _NBUF + b
            bp = (b - 1) % _NBUF
            @pl.when(j >= 1)
            def _():
                write_wait(j - 1, bp)

            jn = j + _NBUF - 1
            @pl.when(jn < _NCHUNK)
            def _():
                gather_start(jn, bp)

            gather_wait(b)
            write_start(j, b)
        return ()

    lax.fori_loop(0, _OUTER, outer, (), unroll=False)
    write_wait(_NCHUNK - 1, (_NCHUNK - 1) % _NBUF)


@jax.jit
def kernel(x, edge_index):
    idx = edge_index.T.reshape(_NW, _NCHUNK, _C)
    grid = plsc.VectorSubcoreMesh(
        core_axis_name="c", subcore_axis_name="s",
        num_cores=_NC, num_subcores=_NS)
    out = pl.kernel(
        _gather_body,
        out_type=jax.ShapeDtypeStruct((_B, _D), jnp.float32),
        mesh=grid,
        scratch_types=(
            [pltpu.VMEM((_NCHUNK, _C), jnp.int32),
             pltpu.VMEM((_NBUF, _C, _D), jnp.float32)]
            + [pltpu.SemaphoreType.DMA] * (2 * _NBUF)
        ),
    )(x, idx)
    return out.reshape(_N_EDGES, 2 * _D)


# speedup vs baseline: 2.3803x; 1.0162x over previous
import functools

import jax
import jax.numpy as jnp
from jax import lax
from jax.experimental import pallas as pl
from jax.experimental.pallas import tpu as pltpu
from jax.experimental.pallas import tpu_sc as plsc

_NC = 2
_NS = 16
_NW = _NC * _NS

_N_NODES = 10000
_D = 128
_N_EDGES = 320000
_B = 2 * _N_EDGES
_PER_W = _B // _NW
_C = 80
_NCHUNK = _PER_W // _C
_NBUF = 10
_OUTER = _NCHUNK // _NBUF


def _gather_body(x_hbm, idx_hbm, out_hbm, idx_v, buf, *sems):
    gsems = sems[:_NBUF]
    wsems = sems[_NBUF:]
    wid = lax.axis_index("s") * _NC + lax.axis_index("c")
    base = wid * _PER_W

    pltpu.sync_copy(idx_hbm.at[wid], idx_v)

    def gather_start(j, b):
        pltpu.async_copy(x_hbm.at[idx_v.at[pl.ds(j * _C, _C)]], buf.at[b],
                         gsems[b])

    def gather_wait(b):
        pltpu.make_async_copy(x_hbm, buf.at[b], gsems[b]).wait()

    def write_start(j, b):
        pltpu.async_copy(buf.at[b], out_hbm.at[pl.ds(base + j * _C, _C)],
                         wsems[b])

    def write_wait(j, b):
        pltpu.make_async_copy(buf.at[b],
                              out_hbm.at[pl.ds(base + j * _C, _C)],
                              wsems[b]).wait()

    for j in range(_NBUF - 1):
        gather_start(j, j)

    def outer(jh, _):
        for b in range(_NBUF):
            j = jh * _NBUF + b
            bp = (b - 1) % _NBUF
            @pl.when(j >= 1)
            def _():
                write_wait(j - 1, bp)

            jn = j + _NBUF - 1
            @pl.when(jn < _NCHUNK)
            def _():
                gather_start(jn, bp)

            gather_wait(b)
            write_start(j, b)
        return ()

    lax.fori_loop(0, _OUTER, outer, (), unroll=False)
    write_wait(_NCHUNK - 1, (_NCHUNK - 1) % _NBUF)


@jax.jit
def kernel(x, edge_index):
    idx = edge_index.T.reshape(_NW, _PER_W)
    grid = plsc.VectorSubcoreMesh(
        core_axis_name="c", subcore_axis_name="s",
        num_cores=_NC, num_subcores=_NS)
    out = pl.kernel(
        _gather_body,
        out_type=jax.ShapeDtypeStruct((_B, _D), jnp.float32),
        mesh=grid,
        scratch_types=(
            [pltpu.VMEM((_PER_W,), jnp.int32),
             pltpu.VMEM((_NBUF, _C, _D), jnp.float32)]
            + [pltpu.SemaphoreType.DMA] * (2 * _NBUF)
        ),
    )(x, idx)
    return out.reshape(_N_EDGES, 2 * _D)


# speedup vs baseline: 7.3723x; 3.0972x over previous
import jax
import jax.numpy as jnp
from jax import lax
from jax.experimental import pallas as pl
from jax.experimental.pallas import tpu as pltpu
from jax.experimental.pallas import tpu_sc as plsc

_NC = 2
_NS = 16
_NW = _NC * _NS

_D = 128
_D2 = 2 * _D
_N_EDGES = 320000
_EP = _N_EDGES // _NW
_C = 80
_NCHUNK = _EP // _C
_NBUF = 5
_OUTER = _NCHUNK // _NBUF


def _gather_body(x_hbm, eidx_hbm, out_hbm, src_v, dst_v, buf, *sems):
    gsems = sems[:_NBUF]
    wsems = sems[_NBUF:]
    wid = lax.axis_index("s") * _NC + lax.axis_index("c")
    ebase = wid * _EP

    pltpu.sync_copy(eidx_hbm.at[pl.ds(ebase, _EP)], src_v)
    pltpu.sync_copy(eidx_hbm.at[pl.ds(_N_EDGES + ebase, _EP)], dst_v)

    def gather_start(j, b):
        sl = pl.ds(j * _C, _C)
        pltpu.async_copy(x_hbm.at[src_v.at[sl]],
                         buf.at[b, pl.ds(0, _C), pl.ds(0, _D)], gsems[b])
        pltpu.async_copy(x_hbm.at[dst_v.at[sl]],
                         buf.at[b, pl.ds(0, _C), pl.ds(_D, _D)], gsems[b])

    def gather_wait(b):
        pltpu.make_async_copy(x_hbm, buf.at[b], gsems[b]).wait()

    def write_start(j, b):
        pltpu.async_copy(buf.at[b], out_hbm.at[pl.ds(ebase + j * _C, _C)],
                         wsems[b])

    def write_wait(j, b):
        pltpu.make_async_copy(buf.at[b],
                              out_hbm.at[pl.ds(ebase + j * _C, _C)],
                              wsems[b]).wait()

    for j in range(_NBUF - 1):
        gather_start(j, j)

    def outer(jh, _):
        for b in range(_NBUF):
            j = jh * _NBUF + b
            bp = (b - 1) % _NBUF
            @pl.when(j >= 1)
            def _():
                write_wait(j - 1, bp)

            jn = j + _NBUF - 1
            @pl.when(jn < _NCHUNK)
            def _():
                gather_start(jn, bp)

            gather_wait(b)
            write_start(j, b)
        return ()

    lax.fori_loop(0, _OUTER, outer, (), unroll=False)
    write_wait(_NCHUNK - 1, (_NCHUNK - 1) % _NBUF)


@jax.jit
def kernel(x, edge_index):
    eidx = edge_index.reshape(-1)
    grid = plsc.VectorSubcoreMesh(
        core_axis_name="c", subcore_axis_name="s",
        num_cores=_NC, num_subcores=_NS)
    return pl.kernel(
        _gather_body,
        out_type=jax.ShapeDtypeStruct((_N_EDGES, _D2), jnp.float32),
        mesh=grid,
        scratch_types=(
            [pltpu.VMEM((_EP,), jnp.int32),
             pltpu.VMEM((_EP,), jnp.int32),
             pltpu.VMEM((_NBUF, _C, _D2), jnp.float32)]
            + [pltpu.SemaphoreType.DMA] * (2 * _NBUF)
        ),
    )(x, eidx)
